# baseline (speedup 1.0000x reference)
import functools

import jax
import jax.numpy as jnp
from jax import lax
from jax.experimental import pallas as pl
from jax.experimental.pallas import tpu as pltpu
from jax.experimental.pallas import tpu_sc as plsc


@functools.lru_cache(maxsize=None)
def _make_trepack(V, D, R):
    grid = (V + R - 1) // R

    def body(t_ref, o_ref):
        x = t_ref[...]
        y = x.T
        z = y.reshape(R // 2, 2, D)
        o_ref[...] = jnp.concatenate([z[:, 0, :], z[:, 1, :]], axis=1)

    return pl.pallas_call(
        body,
        grid=(grid,),
        in_specs=[pl.BlockSpec((D, R), lambda i: (0, i))],
        out_specs=pl.BlockSpec((R // 2, 2 * D), lambda i: (i, 0)),
        out_shape=jax.ShapeDtypeStruct((V // 2, 2 * D), jnp.float32),
    )


@functools.lru_cache(maxsize=None)
def _make_gather(N, D, NC, NS, C, NBUF):
    NW = NC * NS
    n_per_w = N // NW
    n_chunks = n_per_w // C
    assert n_chunks % NBUF == 0
    rounds = n_chunks // NBUF
    mesh = plsc.VectorSubcoreMesh(core_axis_name="c", subcore_axis_name="s")

    @functools.partial(
        pl.kernel,
        mesh=mesh,
        compiler_params=pltpu.CompilerParams(use_tc_tiling_on_sc=False),
        out_type=jax.ShapeDtypeStruct((N, D), jnp.float32),
        scratch_types=[
            pltpu.VMEM((n_chunks, C), jnp.int32),
            *[pltpu.VMEM((C, D), jnp.float32) for _ in range(NBUF)],
            *[pltpu.SemaphoreType.DMA for _ in range(2 * NBUF)],
        ],
    )
    def k(idx_hbm, table_hbm, out_hbm, idx_v, *rest):
        bufs = rest[:NBUF]
        gsems = rest[NBUF:2 * NBUF]
        wsems = rest[2 * NBUF:]
        wid = lax.axis_index("s") * NC + lax.axis_index("c")
        base = wid * n_per_w
        pltpu.sync_copy(idx_hbm.at[wid], idx_v)

        def fire_gather(j, b):
            pltpu.async_copy(table_hbm.at[idx_v.at[j]], bufs[b], gsems[b])

        def fire_write(j, b):
            pltpu.async_copy(bufs[b], out_hbm.at[pl.ds(base + j * C, C)],
                             wsems[b])

        for b in range(NBUF):
            fire_gather(b, b)

        def body(g, carry):
            for b in range(NBUF):
                j = g * NBUF + b
                pltpu.make_async_copy(
                    table_hbm.at[idx_v.at[j]], bufs[b], gsems[b]).wait()
                fire_write(j, b)
            for b in range(NBUF):
                j = g * NBUF + b
                pltpu.make_async_copy(
                    bufs[b], out_hbm.at[pl.ds(base + j * C, C)],
                    wsems[b]).wait()

                @pl.when(g + 1 < rounds)
                def _(b=b):
                    fire_gather((g + 1) * NBUF + b, b)
            return carry

        lax.fori_loop(0, rounds, body, 0)

    return k


def kernel(input, table):
    B, H = input.shape
    V, D = table.shape
    N = B * H
    info = plsc.get_sparse_core_info()
    NC, NS = info.num_cores, info.num_subcores
    C = 128
    NBUF = 5
    idx = input.reshape(NC * NS, N // (NC * NS) // C, C)
    t2 = _make_trepack(V, D, 512)(table.T)
    t3 = t2.reshape(V, D)
    out = _make_gather(N, D, NC, NS, C, NBUF)(idx, t3)
    return out.reshape(B, H, D)


# speedup vs baseline: 2.2121x; 2.2121x over previous
import functools

import jax
import jax.numpy as jnp
from jax import lax
from jax.experimental import pallas as pl
from jax.experimental.pallas import tpu as pltpu
from jax.experimental.pallas import tpu_sc as plsc

_R = 2048


@functools.lru_cache(maxsize=None)
def _make_trepack(V, D, R):
    grid = (V + R - 1) // R
    Vp = grid * R

    def body(t_ref, o_ref):
        x = t_ref[...]
        eye = jnp.eye(D, dtype=jnp.float32)
        y = lax.dot_general(x, eye, (((0,), (0,)), ((), ())),
                            preferred_element_type=jnp.float32)
        o_ref[:, 0:D] = y[0:R // 2]
        o_ref[:, D:2 * D] = y[R // 2:]

    return pl.pallas_call(
        body,
        grid=(grid,),
        in_specs=[pl.BlockSpec((D, R), lambda i: (0, i))],
        out_specs=pl.BlockSpec((R // 2, 2 * D), lambda i: (i, 0)),
        out_shape=jax.ShapeDtypeStruct((Vp // 2, 2 * D), jnp.float32),
    )


@functools.lru_cache(maxsize=None)
def _make_gather(N, Vp, D, NC, NS, C, NBUF, R):
    NW = NC * NS
    n_per_w = N // NW
    n_chunks = n_per_w // C
    assert n_chunks % NBUF == 0
    rounds = n_chunks // NBUF
    half = R // 2
    mesh = plsc.VectorSubcoreMesh(core_axis_name="c", subcore_axis_name="s")

    @functools.partial(
        pl.kernel,
        mesh=mesh,
        compiler_params=pltpu.CompilerParams(use_tc_tiling_on_sc=False),
        out_type=jax.ShapeDtypeStruct((N, D), jnp.float32),
        scratch_types=[
            pltpu.VMEM((n_chunks, C), jnp.int32),
            *[pltpu.VMEM((C, D), jnp.float32) for _ in range(NBUF)],
            *[pltpu.SemaphoreType.DMA for _ in range(2 * NBUF)],
        ],
    )
    def k(idx_hbm, table_hbm, out_hbm, idx_v, *rest):
        bufs = rest[:NBUF]
        gsems = rest[NBUF:2 * NBUF]
        wsems = rest[2 * NBUF:]
        wid = lax.axis_index("s") * NC + lax.axis_index("c")
        base = wid * n_per_w
        pltpu.sync_copy(idx_hbm.at[wid], idx_v)

        def remap(j, carry):
            row = idx_v.at[j]
            for g in range(C // 16):
                sl = pl.ds(g * 16, 16)
                v = row[sl]
                t = (v >> (half.bit_length() - 1)) & 1
                row[sl] = (v & ~(R - 1)) + ((v & (R - 1)) << 1) - t * (R - 1)
            return carry

        lax.fori_loop(0, n_chunks, remap, 0)

        def fire_gather(j, b):
            pltpu.async_copy(table_hbm.at[idx_v.at[j]], bufs[b], gsems[b])

        def fire_write(j, b):
            pltpu.async_copy(bufs[b], out_hbm.at[pl.ds(base + j * C, C)],
                             wsems[b])

        for b in range(NBUF):
            fire_gather(b, b)

        def body(g, carry):
            for b in range(NBUF):
                j = g * NBUF + b
                pltpu.make_async_copy(
                    table_hbm.at[idx_v.at[j]], bufs[b], gsems[b]).wait()
                fire_write(j, b)
            for b in range(NBUF):
                j = g * NBUF + b
                pltpu.make_async_copy(
                    bufs[b], out_hbm.at[pl.ds(base + j * C, C)],
                    wsems[b]).wait()

                @pl.when(g + 1 < rounds)
                def _(b=b):
                    fire_gather((g + 1) * NBUF + b, b)
            return carry

        lax.fori_loop(0, rounds, body, 0)

    return k


def kernel(input, table):
    B, H = input.shape
    V, D = table.shape
    N = B * H
    info = plsc.get_sparse_core_info()
    NC, NS = info.num_cores, info.num_subcores
    C = 128
    NBUF = 5
    idx = input.reshape(NC * NS, N // (NC * NS) // C, C)
    t2 = _make_trepack(V, D, _R)(table.T)
    Vp = t2.shape[0] * 2
    t3 = t2.reshape(Vp, D)
    out = _make_gather(N, Vp, D, NC, NS, C, NBUF, _R)(idx, t3)
    return out.reshape(B, H, D)


# speedup vs baseline: 2.7967x; 1.2642x over previous
import functools

import jax
import jax.numpy as jnp
from jax import lax
from jax.experimental import pallas as pl
from jax.experimental.pallas import tpu as pltpu
from jax.experimental.pallas import tpu_sc as plsc

_R = 4096


@functools.lru_cache(maxsize=None)
def _make_trepack(V, D, R):
    grid = (V + R - 1) // R
    Vp = grid * R

    def body(t_ref, o_ref):
        x = t_ref[...]
        y = x.T
        o_ref[:, 0:D] = y[0:R // 2]
        o_ref[:, D:2 * D] = y[R // 2:]

    return pl.pallas_call(
        body,
        grid=(grid,),
        in_specs=[pl.BlockSpec((D, R), lambda i: (0, i))],
        out_specs=pl.BlockSpec((R // 2, 2 * D), lambda i: (i, 0)),
        out_shape=jax.ShapeDtypeStruct((Vp // 2, 2 * D), jnp.float32),
    )


@functools.lru_cache(maxsize=None)
def _make_gather(N, Vp, D, NC, NS, C, NBUF, R):
    NW = NC * NS
    n_per_w = N // NW
    n_chunks = n_per_w // C
    assert n_chunks % NBUF == 0
    rounds = n_chunks // NBUF
    half = R // 2
    mesh = plsc.VectorSubcoreMesh(core_axis_name="c", subcore_axis_name="s")

    @functools.partial(
        pl.kernel,
        mesh=mesh,
        compiler_params=pltpu.CompilerParams(use_tc_tiling_on_sc=False),
        out_type=jax.ShapeDtypeStruct((N, D), jnp.float32),
        scratch_types=[
            pltpu.VMEM((n_chunks, C), jnp.int32),
            *[pltpu.VMEM((C, D), jnp.float32) for _ in range(NBUF)],
            *[pltpu.SemaphoreType.DMA for _ in range(2 * NBUF)],
        ],
    )
    def k(idx_hbm, table_hbm, out_hbm, idx_v, *rest):
        bufs = rest[:NBUF]
        gsems = rest[NBUF:2 * NBUF]
        wsems = rest[2 * NBUF:]
        wid = lax.axis_index("s") * NC + lax.axis_index("c")
        base = wid * n_per_w
        pltpu.sync_copy(idx_hbm.at[wid], idx_v)

        def remap(j, carry):
            row = idx_v.at[j]
            for g in range(C // 16):
                sl = pl.ds(g * 16, 16)
                v = row[sl]
                t = (v >> (half.bit_length() - 1)) & 1
                row[sl] = (v & ~(R - 1)) + ((v & (R - 1)) << 1) - t * (R - 1)
            return carry

        lax.fori_loop(0, n_chunks, remap, 0)

        def fire_gather(j, b):
            pltpu.async_copy(table_hbm.at[idx_v.at[j]], bufs[b], gsems[b])

        def fire_write(j, b):
            pltpu.async_copy(bufs[b], out_hbm.at[pl.ds(base + j * C, C)],
                             wsems[b])

        for b in range(NBUF):
            fire_gather(b, b)

        def body(g, carry):
            for b in range(NBUF):
                j = g * NBUF + b
                pltpu.make_async_copy(
                    table_hbm.at[idx_v.at[j]], bufs[b], gsems[b]).wait()
                fire_write(j, b)
            for b in range(NBUF):
                j = g * NBUF + b
                pltpu.make_async_copy(
                    bufs[b], out_hbm.at[pl.ds(base + j * C, C)],
                    wsems[b]).wait()

                @pl.when(g + 1 < rounds)
                def _(b=b):
                    fire_gather((g + 1) * NBUF + b, b)
            return carry

        lax.fori_loop(0, rounds, body, 0)

    return k


def kernel(input, table):
    B, H = input.shape
    V, D = table.shape
    N = B * H
    info = plsc.get_sparse_core_info()
    NC, NS = info.num_cores, info.num_subcores
    C = 128
    NBUF = 5
    idx = input.reshape(NC * NS, N // (NC * NS) // C, C)
    t2 = _make_trepack(V, D, _R)(table.T)
    Vp = t2.shape[0] * 2
    t3 = t2.reshape(Vp, D)
    out = _make_gather(N, Vp, D, NC, NS, C, NBUF, _R)(idx, t3)
    return out.reshape(B, H, D)


# speedup vs baseline: 3.2159x; 1.1499x over previous
import functools

import jax
import jax.numpy as jnp
from jax import lax
from jax.experimental import pallas as pl
from jax.experimental.pallas import tpu as pltpu
from jax.experimental.pallas import tpu_sc as plsc

_R = 8192


@functools.lru_cache(maxsize=None)
def _make_trepack(V, D, R):
    grid = (V + R - 1) // R
    Vp = grid * R

    def body(t_ref, o_ref):
        x = t_ref[...]
        y = x.T
        o_ref[:, 0:D] = y[0:R // 2]
        o_ref[:, D:2 * D] = y[R // 2:]

    return pl.pallas_call(
        body,
        grid=(grid,),
        in_specs=[pl.BlockSpec((D, R), lambda i: (0, i))],
        out_specs=pl.BlockSpec((R // 2, 2 * D), lambda i: (i, 0)),
        out_shape=jax.ShapeDtypeStruct((Vp // 2, 2 * D), jnp.float32),
    )


@functools.lru_cache(maxsize=None)
def _make_gather(N, Vp, D, NC, NS, C, NBUF, R):
    NW = NC * NS
    n_per_w = N // NW
    n_chunks = n_per_w // C
    assert n_chunks % NBUF == 0
    rounds = n_chunks // NBUF
    half = R // 2
    mesh = plsc.VectorSubcoreMesh(core_axis_name="c", subcore_axis_name="s")

    @functools.partial(
        pl.kernel,
        mesh=mesh,
        compiler_params=pltpu.CompilerParams(use_tc_tiling_on_sc=False),
        out_type=jax.ShapeDtypeStruct((N, D), jnp.float32),
        scratch_types=[
            pltpu.VMEM((n_chunks, C), jnp.int32),
            *[pltpu.VMEM((C, D), jnp.float32) for _ in range(NBUF)],
            *[pltpu.SemaphoreType.DMA for _ in range(2 * NBUF)],
        ],
    )
    def k(idx_hbm, table_hbm, out_hbm, idx_v, *rest):
        bufs = rest[:NBUF]
        gsems = rest[NBUF:2 * NBUF]
        wsems = rest[2 * NBUF:]
        wid = lax.axis_index("s") * NC + lax.axis_index("c")
        base = wid * n_per_w
        pltpu.sync_copy(idx_hbm.at[wid], idx_v)

        def remap(j, carry):
            row = idx_v.at[j]
            for g in range(C // 16):
                sl = pl.ds(g * 16, 16)
                v = row[sl]
                t = (v >> (half.bit_length() - 1)) & 1
                row[sl] = (v & ~(R - 1)) + ((v & (R - 1)) << 1) - t * (R - 1)
            return carry

        lax.fori_loop(0, n_chunks, remap, 0)

        def fire_gather(j, b):
            pltpu.async_copy(table_hbm.at[idx_v.at[j]], bufs[b], gsems[b])

        def fire_write(j, b):
            pltpu.async_copy(bufs[b], out_hbm.at[pl.ds(base + j * C, C)],
                             wsems[b])

        for b in range(NBUF):
            fire_gather(b, b)

        def body(g, carry):
            for b in range(NBUF):
                j = g * NBUF + b
                pltpu.make_async_copy(
                    table_hbm.at[idx_v.at[j]], bufs[b], gsems[b]).wait()
                fire_write(j, b)
            for b in range(NBUF):
                j = g * NBUF + b
                pltpu.make_async_copy(
                    bufs[b], out_hbm.at[pl.ds(base + j * C, C)],
                    wsems[b]).wait()

                @pl.when(g + 1 < rounds)
                def _(b=b):
                    fire_gather((g + 1) * NBUF + b, b)
            return carry

        lax.fori_loop(0, rounds, body, 0)

    return k


def kernel(input, table):
    B, H = input.shape
    V, D = table.shape
    N = B * H
    info = plsc.get_sparse_core_info()
    NC, NS = info.num_cores, info.num_subcores
    C = 128
    NBUF = 5
    idx = input.reshape(NC * NS, N // (NC * NS) // C, C)
    t2 = _make_trepack(V, D, _R)(table.T)
    Vp = t2.shape[0] * 2
    t3 = t2.reshape(Vp, D)
    out = _make_gather(N, Vp, D, NC, NS, C, NBUF, _R)(idx, t3)
    return out.reshape(B, H, D)


# speedup vs baseline: 3.4647x; 1.0774x over previous
import functools

import jax
import jax.numpy as jnp
from jax import lax
from jax.experimental import pallas as pl
from jax.experimental.pallas import tpu as pltpu
from jax.experimental.pallas import tpu_sc as plsc

_R = 16384


@functools.lru_cache(maxsize=None)
def _make_trepack(V, D, R):
    grid = (V + R - 1) // R
    Vp = grid * R

    def body(t_ref, o_ref):
        x = t_ref[...]
        y = x.T
        o_ref[:, 0:D] = y[0:R // 2]
        o_ref[:, D:2 * D] = y[R // 2:]

    return pl.pallas_call(
        body,
        grid=(grid,),
        in_specs=[pl.BlockSpec((D, R), lambda i: (0, i))],
        out_specs=pl.BlockSpec((R // 2, 2 * D), lambda i: (i, 0)),
        out_shape=jax.ShapeDtypeStruct((Vp // 2, 2 * D), jnp.float32),
    )


@functools.lru_cache(maxsize=None)
def _make_gather(N, Vp, D, NC, NS, C, NBUF, R):
    NW = NC * NS
    n_per_w = N // NW
    n_chunks = n_per_w // C
    assert n_chunks % NBUF == 0
    rounds = n_chunks // NBUF
    half = R // 2
    mesh = plsc.VectorSubcoreMesh(core_axis_name="c", subcore_axis_name="s")

    @functools.partial(
        pl.kernel,
        mesh=mesh,
        compiler_params=pltpu.CompilerParams(use_tc_tiling_on_sc=False),
        out_type=jax.ShapeDtypeStruct((N, D), jnp.float32),
        scratch_types=[
            pltpu.VMEM((n_chunks, C), jnp.int32),
            *[pltpu.VMEM((C, D), jnp.float32) for _ in range(NBUF)],
            *[pltpu.SemaphoreType.DMA for _ in range(2 * NBUF)],
        ],
    )
    def k(idx_hbm, table_hbm, out_hbm, idx_v, *rest):
        bufs = rest[:NBUF]
        gsems = rest[NBUF:2 * NBUF]
        wsems = rest[2 * NBUF:]
        wid = lax.axis_index("s") * NC + lax.axis_index("c")
        base = wid * n_per_w
        pltpu.sync_copy(idx_hbm.at[wid], idx_v)

        def remap(j, carry):
            row = idx_v.at[j]
            for g in range(C // 16):
                sl = pl.ds(g * 16, 16)
                v = row[sl]
                t = (v >> (half.bit_length() - 1)) & 1
                row[sl] = (v & ~(R - 1)) + ((v & (R - 1)) << 1) - t * (R - 1)
            return carry

        lax.fori_loop(0, n_chunks, remap, 0)

        def fire_gather(j, b):
            pltpu.async_copy(table_hbm.at[idx_v.at[j]], bufs[b], gsems[b])

        def fire_write(j, b):
            pltpu.async_copy(bufs[b], out_hbm.at[pl.ds(base + j * C, C)],
                             wsems[b])

        for b in range(NBUF):
            fire_gather(b, b)

        def body(g, carry):
            for b in range(NBUF):
                j = g * NBUF + b
                pltpu.make_async_copy(
                    table_hbm.at[idx_v.at[j]], bufs[b], gsems[b]).wait()
                fire_write(j, b)
            for b in range(NBUF):
                j = g * NBUF + b
                pltpu.make_async_copy(
                    bufs[b], out_hbm.at[pl.ds(base + j * C, C)],
                    wsems[b]).wait()

                @pl.when(g + 1 < rounds)
                def _(b=b):
                    fire_gather((g + 1) * NBUF + b, b)
            return carry

        lax.fori_loop(0, rounds, body, 0)

    return k


def kernel(input, table):
    B, H = input.shape
    V, D = table.shape
    N = B * H
    info = plsc.get_sparse_core_info()
    NC, NS = info.num_cores, info.num_subcores
    C = 128
    NBUF = 5
    idx = input.reshape(NC * NS, N // (NC * NS) // C, C)
    t2 = _make_trepack(V, D, _R)(table.T)
    Vp = t2.shape[0] * 2
    t3 = t2.reshape(Vp, D)
    out = _make_gather(N, Vp, D, NC, NS, C, NBUF, _R)(idx, t3)
    return out.reshape(B, H, D)


# speedup vs baseline: 3.5997x; 1.0390x over previous
import functools

import jax
import jax.numpy as jnp
from jax import lax
from jax.experimental import pallas as pl
from jax.experimental.pallas import tpu as pltpu
from jax.experimental.pallas import tpu_sc as plsc

_R = 32768


@functools.lru_cache(maxsize=None)
def _make_trepack(V, D, R):
    grid = (V + R - 1) // R
    Vp = grid * R

    def body(t_ref, o_ref):
        x = t_ref[...]
        y = x.T
        o_ref[:, 0:D] = y[0:R // 2]
        o_ref[:, D:2 * D] = y[R // 2:]

    return pl.pallas_call(
        body,
        grid=(grid,),
        in_specs=[pl.BlockSpec((D, R), lambda i: (0, i))],
        out_specs=pl.BlockSpec((R // 2, 2 * D), lambda i: (i, 0)),
        out_shape=jax.ShapeDtypeStruct((Vp // 2, 2 * D), jnp.float32),
    )


@functools.lru_cache(maxsize=None)
def _make_gather(N, Vp, D, NC, NS, C, NBUF, R):
    NW = NC * NS
    n_per_w = N // NW
    n_chunks = n_per_w // C
    assert n_chunks % NBUF == 0
    rounds = n_chunks // NBUF
    half = R // 2
    mesh = plsc.VectorSubcoreMesh(core_axis_name="c", subcore_axis_name="s")

    @functools.partial(
        pl.kernel,
        mesh=mesh,
        compiler_params=pltpu.CompilerParams(use_tc_tiling_on_sc=False),
        out_type=jax.ShapeDtypeStruct((N, D), jnp.float32),
        scratch_types=[
            pltpu.VMEM((n_chunks, C), jnp.int32),
            *[pltpu.VMEM((C, D), jnp.float32) for _ in range(NBUF)],
            *[pltpu.SemaphoreType.DMA for _ in range(2 * NBUF)],
        ],
    )
    def k(idx_hbm, table_hbm, out_hbm, idx_v, *rest):
        bufs = rest[:NBUF]
        gsems = rest[NBUF:2 * NBUF]
        wsems = rest[2 * NBUF:]
        wid = lax.axis_index("s") * NC + lax.axis_index("c")
        base = wid * n_per_w
        pltpu.sync_copy(idx_hbm.at[wid], idx_v)

        def remap(j, carry):
            row = idx_v.at[j]
            for g in range(C // 16):
                sl = pl.ds(g * 16, 16)
                v = row[sl]
                t = (v >> (half.bit_length() - 1)) & 1
                row[sl] = (v & ~(R - 1)) + ((v & (R - 1)) << 1) - t * (R - 1)
            return carry

        lax.fori_loop(0, n_chunks, remap, 0)

        def fire_gather(j, b):
            pltpu.async_copy(table_hbm.at[idx_v.at[j]], bufs[b], gsems[b])

        def fire_write(j, b):
            pltpu.async_copy(bufs[b], out_hbm.at[pl.ds(base + j * C, C)],
                             wsems[b])

        for b in range(NBUF):
            fire_gather(b, b)

        def body(g, carry):
            for b in range(NBUF):
                j = g * NBUF + b
                pltpu.make_async_copy(
                    table_hbm.at[idx_v.at[j]], bufs[b], gsems[b]).wait()
                fire_write(j, b)
            for b in range(NBUF):
                j = g * NBUF + b
                pltpu.make_async_copy(
                    bufs[b], out_hbm.at[pl.ds(base + j * C, C)],
                    wsems[b]).wait()

                @pl.when(g + 1 < rounds)
                def _(b=b):
                    fire_gather((g + 1) * NBUF + b, b)
            return carry

        lax.fori_loop(0, rounds, body, 0)

    return k


def kernel(input, table):
    B, H = input.shape
    V, D = table.shape
    N = B * H
    info = plsc.get_sparse_core_info()
    NC, NS = info.num_cores, info.num_subcores
    C = 128
    NBUF = 5
    idx = input.reshape(NC * NS, N // (NC * NS) // C, C)
    t2 = _make_trepack(V, D, _R)(table.T)
    Vp = t2.shape[0] * 2
    t3 = t2.reshape(Vp, D)
    out = _make_gather(N, Vp, D, NC, NS, C, NBUF, _R)(idx, t3)
    return out.reshape(B, H, D)
